# Initial kernel scaffold; baseline (speedup 1.0000x reference)
import functools

import jax
import jax.numpy as jnp
from jax import lax
from jax.experimental import pallas as pl
from jax.experimental.pallas import tpu as pltpu
from jax.experimental.pallas import tpu_sc as plsc

_info = plsc.get_sparse_core_info()
_NC, _NS = _info.num_cores, _info.num_subcores
_NW = _NC * _NS


def _make_gather(vocab: int, dim: int, total: int, chunk: int):
    assert total % (_NW * chunk) == 0
    per_w = total // _NW
    n_chunks = per_w // chunk
    assert n_chunks >= 2 and n_chunks % 2 == 0
    mesh = plsc.VectorSubcoreMesh(core_axis_name="c", subcore_axis_name="s")

    @functools.partial(
        pl.kernel,
        mesh=mesh,
        out_type=jax.ShapeDtypeStruct((total, dim), jnp.float32),
        scratch_types=[
            pltpu.VMEM((per_w,), jnp.int32),
            pltpu.VMEM((chunk, dim), jnp.float32),
            pltpu.VMEM((chunk, dim), jnp.float32),
            pltpu.SemaphoreType.DMA,
            pltpu.SemaphoreType.DMA,
        ],
    )
    def gather_kernel(idx_hbm, tab_hbm, out_hbm, idx_v, rows0, rows1, sem0, sem1):
        wid = lax.axis_index("s") * _NC + lax.axis_index("c")
        base = wid * per_w
        pltpu.sync_copy(idx_hbm.at[pl.ds(base, per_w)], idx_v)

        rows = (rows0, rows1)
        sems = (sem0, sem1)

        pltpu.async_copy(tab_hbm.at[idx_v.at[pl.ds(0, chunk)]], rows0, sem0)

        @pl.loop(0, n_chunks - 2, step=2)
        def _(o):
            for b in range(2):
                g = o + b
                nb = (b + 1) % 2
                pltpu.async_copy(
                    tab_hbm.at[idx_v.at[pl.ds((g + 1) * chunk, chunk)]],
                    rows[nb],
                    sems[nb],
                )
                pltpu.make_async_copy(
                    tab_hbm.at[idx_v.at[pl.ds(0, chunk)]], rows[b], sems[b]
                ).wait()
                pltpu.sync_copy(rows[b], out_hbm.at[pl.ds(base + g * chunk, chunk)])

        g = n_chunks - 2
        pltpu.async_copy(
            tab_hbm.at[idx_v.at[pl.ds((g + 1) * chunk, chunk)]], rows1, sem1
        )
        pltpu.make_async_copy(
            tab_hbm.at[idx_v.at[pl.ds(0, chunk)]], rows0, sem0
        ).wait()
        pltpu.sync_copy(rows0, out_hbm.at[pl.ds(base + g * chunk, chunk)])

        g = n_chunks - 1
        pltpu.make_async_copy(
            tab_hbm.at[idx_v.at[pl.ds(0, chunk)]], rows1, sem1
        ).wait()
        pltpu.sync_copy(rows1, out_hbm.at[pl.ds(base + g * chunk, chunk)])

    return gather_kernel


@jax.jit
def kernel(x, emb_t):
    batch, hist = x.shape
    vocab, dim = emb_t.shape
    total = batch * hist
    flat_idx = x.reshape((total,)).astype(jnp.int32)
    out = _make_gather(vocab, dim, total, 512)(flat_idx, emb_t)
    return out.reshape((batch, hist, dim))


# speedup vs baseline: 1.8736x; 1.8736x over previous
import functools

import jax
import jax.numpy as jnp
from jax import lax
from jax.experimental import pallas as pl
from jax.experimental.pallas import tpu as pltpu
from jax.experimental.pallas import tpu_sc as plsc

_info = plsc.get_sparse_core_info()
_NC, _NS = _info.num_cores, _info.num_subcores
_NW = _NC * _NS


def _make_gather(vocab: int, dim: int, total: int, chunk: int):
    assert total % (_NW * chunk) == 0
    per_w = total // _NW
    n_chunks = per_w // chunk
    assert n_chunks >= 2 and n_chunks % 2 == 0
    mesh = plsc.VectorSubcoreMesh(core_axis_name="c", subcore_axis_name="s")

    @functools.partial(
        pl.kernel,
        mesh=mesh,
        out_type=jax.ShapeDtypeStruct((total, dim), jnp.float32),
        scratch_types=[
            pltpu.VMEM((per_w,), jnp.int32),
            pltpu.VMEM((chunk, dim), jnp.float32),
            pltpu.VMEM((chunk, dim), jnp.float32),
            pltpu.SemaphoreType.DMA,
            pltpu.SemaphoreType.DMA,
        ],
        compiler_params=pltpu.CompilerParams(use_tc_tiling_on_sc=False),
    )
    def gather_kernel(idx_hbm, tab_hbm, out_hbm, idx_v, rows0, rows1, sem0, sem1):
        wid = lax.axis_index("s") * _NC + lax.axis_index("c")
        base = wid * per_w
        pltpu.sync_copy(idx_hbm.at[pl.ds(base, per_w)], idx_v)

        rows = (rows0, rows1)
        sems = (sem0, sem1)

        pltpu.async_copy(tab_hbm.at[idx_v.at[pl.ds(0, chunk)]], rows0, sem0)

        @pl.loop(0, n_chunks - 2, step=2)
        def _(o):
            for b in range(2):
                g = o + b
                nb = (b + 1) % 2
                pltpu.async_copy(
                    tab_hbm.at[idx_v.at[pl.ds((g + 1) * chunk, chunk)]],
                    rows[nb],
                    sems[nb],
                )
                pltpu.make_async_copy(
                    tab_hbm.at[idx_v.at[pl.ds(0, chunk)]], rows[b], sems[b]
                ).wait()
                pltpu.sync_copy(rows[b], out_hbm.at[pl.ds(base + g * chunk, chunk)])

        g = n_chunks - 2
        pltpu.async_copy(
            tab_hbm.at[idx_v.at[pl.ds((g + 1) * chunk, chunk)]], rows1, sem1
        )
        pltpu.make_async_copy(
            tab_hbm.at[idx_v.at[pl.ds(0, chunk)]], rows0, sem0
        ).wait()
        pltpu.sync_copy(rows0, out_hbm.at[pl.ds(base + g * chunk, chunk)])

        g = n_chunks - 1
        pltpu.make_async_copy(
            tab_hbm.at[idx_v.at[pl.ds(0, chunk)]], rows1, sem1
        ).wait()
        pltpu.sync_copy(rows1, out_hbm.at[pl.ds(base + g * chunk, chunk)])

    return gather_kernel


@jax.jit
def kernel(x, emb_t):
    batch, hist = x.shape
    vocab, dim = emb_t.shape
    total = batch * hist
    flat_idx = x.reshape((total,)).astype(jnp.int32)
    out = _make_gather(vocab, dim, total, 512)(flat_idx, emb_t)
    return out.reshape((batch, hist, dim))
